# baseline (speedup 1.0000x reference)
import functools

import jax
import jax.numpy as jnp
from jax import lax
from jax.experimental import pallas as pl
from jax.experimental.pallas import tpu as pltpu
from jax.experimental.pallas import tpu_sc as plsc

N_NODE = 10000
N_PRICE = 100
N_CAT = 1000
EMB = 64

NN_P = 10240
NP_P = 128
NC_P = 1024

NCORES = 2
NSUB = 16
DPT = EMB // NSUB
ECH = 512

ADJ_PC = 626
ADJ_TOT = ADJ_PC * ECH * NCORES
V_PC = 20
V_TOT = V_PC * ECH
GA_PC = 20
GA_TOT = GA_PC * ECH
GB_PC = 4
GB_TOT = GB_PC * ECH

_f32 = jnp.float32
_i32 = jnp.int32


def _sc_body(include_cat, *refs):
    names = [
        "adj_e", "vp_e", "vc_e", "pv_e", "pc_e", "cp_e", "cv_e",
        "emb_t", "pri_t", "cat_t",
        "s_emb", "s_cat", "s_pri", "z_pv", "z_pc", "z_cp", "z_cv", "m_pv",
        "m_pc", "m_cp", "m_cv", "zeros_t", "zeros_g",
        "adjT_out", "vpT_out", "vcT_out", "pvT_out", "pcT_out",
    ] + (["cpT_out", "cvT_out"] if include_cat else []) + [
        "ebuf", "embT_sl", "priT_sl", "catT_sl", "outT", "outG",
        "sbufA", "sbufB", "mbufA", "mbufB", "zbufA", "zbufB",
        "denA", "denB", "wbufA", "wbufB", "esem0", "esem1",
    ]
    r = dict(zip(names, refs, strict=True))

    cid = lax.axis_index("c")
    sid = lax.axis_index("s")

    ebuf = r["ebuf"]
    outT, outG = r["outT"], r["outG"]
    esem = (r["esem0"], r["esem1"])
    dslice = pl.ds(sid * DPT, DPT)

    def e_start(edata, ch, b):
        pltpu.async_copy(edata.at[ch], ebuf.at[b], esem[b])

    def e_wait(edata, ch, b):
        pltpu.make_async_copy(edata.at[ch], ebuf.at[b], esem[b]).wait()

    def edge_loop(edata, cbase, pc, group_fn, unroll):
        for b in range(2):
            e_start(edata, cbase + b, b)

        def body(kk, carry):
            for b in range(2):
                k = kk * 2 + b
                e_wait(edata, cbase + k, b)

                @plsc.parallel_loop(0, ECH // 16, unroll=unroll)
                def _groups(g, b=b, k=k):
                    group_fn(b, k, g)

                @pl.when(k + 2 < pc)
                def _prefetch(b=b, k=k):
                    e_start(edata, cbase + k + 2, b)
            return carry
        lax.fori_loop(0, pc // 2, body, 0)

    def spmm_group(tbl_sl):
        def fn(b, k, g):
            sl = pl.ds(g * 16, 16)
            r16 = ebuf[b, 0, sl]
            c16 = ebuf[b, 1, sl]
            v16 = plsc.bitcast(ebuf[b, 2, sl], _f32)
            for d in range(DPT):
                df = jnp.full((16,), d, _i32)
                gv = plsc.load_gather(tbl_sl, [df, c16])
                plsc.addupdate_scatter(outT, [df, r16], gv * v16)
        return fn

    def weight_group(mref, sref, wbuf, den):
        def fn(b, k, g):
            sl = pl.ds(g * 16, 16)
            r16 = ebuf[b, 0, sl]
            c16 = ebuf[b, 1, sl]
            v16 = plsc.bitcast(ebuf[b, 2, sl], _f32)
            m16 = plsc.load_gather(mref, [r16])
            s16 = plsc.load_gather(sref, [c16])
            w16 = jnp.exp(m16 * s16) * v16
            wbuf[pl.ds(k * ECH + g * 16, 16)] = w16
            plsc.addupdate_scatter(den, [r16], w16)
        return fn

    def gate_group(tbl_sl, wbuf, den, zref):
        def fn(b, k, g):
            sl = pl.ds(g * 16, 16)
            r16 = ebuf[b, 0, sl]
            c16 = ebuf[b, 1, sl]
            w16 = wbuf[pl.ds(k * ECH + g * 16, 16)]
            d16 = plsc.load_gather(den, [r16])
            z16 = plsc.load_gather(zref, [r16])
            u16 = w16 / (d16 + 1e-8 * z16)
            for d in range(DPT):
                df = jnp.full((16,), d, _i32)
                gv = plsc.load_gather(tbl_sl, [df, c16])
                plsc.addupdate_scatter(outG, [df, r16], gv * u16)
        return fn

    def zero_vec(ref, n):
        def zb(i, c):
            ref[pl.ds(i * 16, 16)] = jnp.zeros((16,), _f32)
            return c
        lax.fori_loop(0, n // 16, zb, 0)

    pltpu.sync_copy(r["emb_t"].at[dslice], r["embT_sl"])
    pltpu.sync_copy(r["pri_t"].at[dslice], r["priT_sl"])
    pltpu.sync_copy(r["cat_t"].at[dslice], r["catT_sl"])
    pltpu.sync_copy(r["s_emb"], r["sbufA"])

    @pl.when(cid == 0)
    def _stage0():
        pltpu.sync_copy(r["s_cat"], r["sbufB"])
        pltpu.sync_copy(r["m_pv"], r["mbufA"].at[pl.ds(0, NP_P)])
        pltpu.sync_copy(r["m_pc"], r["mbufB"].at[pl.ds(0, NP_P)])
        pltpu.sync_copy(r["z_pv"], r["zbufA"].at[pl.ds(0, NP_P)])
        pltpu.sync_copy(r["z_pc"], r["zbufB"].at[pl.ds(0, NP_P)])

    @pl.when(cid == 1)
    def _stage1():
        pltpu.sync_copy(r["s_pri"], r["sbufB"].at[pl.ds(0, NP_P)])
        pltpu.sync_copy(r["m_cv"], r["mbufA"])
        pltpu.sync_copy(r["m_cp"], r["mbufB"])
        pltpu.sync_copy(r["z_cv"], r["zbufA"])
        pltpu.sync_copy(r["z_cp"], r["zbufB"])

    pltpu.sync_copy(r["zeros_t"], outT)
    edge_loop(r["adj_e"], cid * ADJ_PC, ADJ_PC, spmm_group(r["embT_sl"]),
              unroll=4)
    pltpu.sync_copy(outT, r["adjT_out"].at[pl.ds(cid * EMB + sid * DPT, DPT)])

    pltpu.sync_copy(r["zeros_t"], outT)

    @pl.when(cid == 0)
    def _vp():
        edge_loop(r["vp_e"], 0, V_PC, spmm_group(r["priT_sl"]), unroll=2)
        pltpu.sync_copy(outT, r["vpT_out"].at[dslice])

    @pl.when(cid == 1)
    def _vc():
        edge_loop(r["vc_e"], 0, V_PC, spmm_group(r["catT_sl"]), unroll=2)
        pltpu.sync_copy(outT, r["vcT_out"].at[dslice])

    zero_vec(r["denA"], NC_P)
    zero_vec(r["denB"], NC_P)

    @pl.when(cid == 0)
    def _gates0():
        edge_loop(r["pv_e"], 0, GA_PC,
                  weight_group(r["mbufA"], r["sbufA"], r["wbufA"], r["denA"]),
                  unroll=1)
        edge_loop(r["pc_e"], 0, GB_PC,
                  weight_group(r["mbufB"], r["sbufB"], r["wbufB"], r["denB"]),
                  unroll=1)
        pltpu.sync_copy(r["zeros_g"], outG)
        edge_loop(r["pv_e"], 0, GA_PC,
                  gate_group(r["embT_sl"], r["wbufA"], r["denA"], r["zbufA"]),
                  unroll=1)
        pltpu.sync_copy(outG, r["pvT_out"].at[dslice])
        pltpu.sync_copy(r["zeros_g"], outG)
        edge_loop(r["pc_e"], 0, GB_PC,
                  gate_group(r["catT_sl"], r["wbufB"], r["denB"], r["zbufB"]),
                  unroll=1)
        pltpu.sync_copy(outG, r["pcT_out"].at[dslice])

    if include_cat:
        @pl.when(cid == 1)
        def _gates1():
            edge_loop(r["cv_e"], 0, GA_PC,
                      weight_group(r["mbufA"], r["sbufA"], r["wbufA"],
                                   r["denA"]), unroll=1)
            edge_loop(r["cp_e"], 0, GB_PC,
                      weight_group(r["mbufB"], r["sbufB"], r["wbufB"],
                                   r["denB"]), unroll=1)
            pltpu.sync_copy(r["zeros_g"], outG)
            edge_loop(r["cv_e"], 0, GA_PC,
                      gate_group(r["embT_sl"], r["wbufA"], r["denA"],
                                 r["zbufA"]), unroll=1)
            pltpu.sync_copy(outG, r["cvT_out"].at[dslice])
            pltpu.sync_copy(r["zeros_g"], outG)
            edge_loop(r["cp_e"], 0, GB_PC,
                      gate_group(r["priT_sl"], r["wbufB"], r["denB"],
                                 r["zbufB"]), unroll=1)
            pltpu.sync_copy(outG, r["cpT_out"].at[dslice])


@functools.cache
def _make_sc_kernel(include_cat: bool):
    out_type = [
        jax.ShapeDtypeStruct((2 * EMB, NN_P), _f32),
        jax.ShapeDtypeStruct((EMB, NN_P), _f32),
        jax.ShapeDtypeStruct((EMB, NN_P), _f32),
        jax.ShapeDtypeStruct((EMB, NC_P), _f32),
        jax.ShapeDtypeStruct((EMB, NC_P), _f32),
    ]
    if include_cat:
        out_type += [
            jax.ShapeDtypeStruct((EMB, NC_P), _f32),
            jax.ShapeDtypeStruct((EMB, NC_P), _f32),
        ]
    scratch = [
        pltpu.VMEM((2, 3, ECH), _i32),
        pltpu.VMEM((DPT, NN_P), _f32),
        pltpu.VMEM((DPT, NP_P), _f32),
        pltpu.VMEM((DPT, NC_P), _f32),
        pltpu.VMEM((DPT, NN_P), _f32),
        pltpu.VMEM((DPT, NC_P), _f32),
        pltpu.VMEM((NN_P,), _f32),
        pltpu.VMEM((NC_P,), _f32),
        pltpu.VMEM((NC_P,), _f32),
        pltpu.VMEM((NC_P,), _f32),
        pltpu.VMEM((NC_P,), _f32),
        pltpu.VMEM((NC_P,), _f32),
        pltpu.VMEM((NC_P,), _f32),
        pltpu.VMEM((NC_P,), _f32),
        pltpu.VMEM((GA_TOT,), _f32),
        pltpu.VMEM((GB_TOT,), _f32),
        pltpu.SemaphoreType.DMA,
        pltpu.SemaphoreType.DMA,
    ]
    mesh = plsc.VectorSubcoreMesh(
        core_axis_name="c", subcore_axis_name="s",
        num_cores=NCORES, num_subcores=NSUB)
    return pl.kernel(
        functools.partial(_sc_body, include_cat),
        out_type=out_type, mesh=mesh, scratch_types=scratch,
        compiler_params=pltpu.CompilerParams(needs_layout_passes=False,
                                             use_tc_tiling_on_sc=False),
        name=f"cohhn_sc_layer_cat{int(include_cat)}")


def _prep_a_body(emb_ref, mpv_ref, mcv_ref, s_ref, zpv_ref, zcv_ref):
    k = pl.program_id(0)
    blk = emb_ref[...]
    s = jnp.sum(blk, axis=1, keepdims=True)
    s_ref[...] = s.reshape(8, 128)
    jrow = lax.broadcasted_iota(_i32, (1024, 1), 0) + k * 1024
    mask = (jrow < N_NODE).astype(_f32)

    @pl.when(k == 0)
    def _init():
        zpv_ref[...] = jnp.zeros_like(zpv_ref)
        zcv_ref[...] = jnp.zeros_like(zcv_ref)

    zpv_ref[...] += jnp.sum(jnp.exp(s * mpv_ref[...]) * mask, axis=0,
                            keepdims=True)
    zcv_ref[...] += jnp.sum(jnp.exp(s * mcv_ref[...]) * mask, axis=0,
                            keepdims=True)


def _prep_a(emb_p, mpv_row, mcv_row):
    return pl.pallas_call(
        _prep_a_body,
        grid=(NN_P // 1024,),
        in_specs=[
            pl.BlockSpec((1024, EMB), lambda k: (k, 0)),
            pl.BlockSpec((1, NP_P), lambda k: (0, 0)),
            pl.BlockSpec((1, NC_P), lambda k: (0, 0)),
        ],
        out_specs=[
            pl.BlockSpec((8, 128), lambda k: (k, 0)),
            pl.BlockSpec((1, NP_P), lambda k: (0, 0)),
            pl.BlockSpec((1, NC_P), lambda k: (0, 0)),
        ],
        out_shape=[
            jax.ShapeDtypeStruct((NN_P // 128, 128), _f32),
            jax.ShapeDtypeStruct((1, NP_P), _f32),
            jax.ShapeDtypeStruct((1, NC_P), _f32),
        ],
    )(emb_p, mpv_row, mcv_row)


def _prep_b_body(cat_ref, pri_ref, mpc_ref, mcp_ref,
                 scat_ref, spri_ref, zpc_ref, zcp_ref):
    scat = jnp.sum(cat_ref[...], axis=1, keepdims=True)
    spri = jnp.sum(pri_ref[...], axis=1, keepdims=True)
    scat_ref[...] = scat.reshape(8, 128)
    spri_ref[...] = spri.reshape(1, 128)
    mc = (lax.broadcasted_iota(_i32, (1024, 1), 0) < N_CAT).astype(_f32)
    mp = (lax.broadcasted_iota(_i32, (128, 1), 0) < N_PRICE).astype(_f32)
    zpc_ref[...] = jnp.sum(jnp.exp(scat * mpc_ref[...]) * mc, axis=0,
                           keepdims=True)
    zcp_ref[...] = jnp.sum(jnp.exp(spri * mcp_ref[...]) * mp, axis=0,
                           keepdims=True)


def _prep_b(cat_p, pri_p, mpc_row, mcp_row):
    return pl.pallas_call(
        _prep_b_body,
        out_shape=[
            jax.ShapeDtypeStruct((NC_P // 128, 128), _f32),
            jax.ShapeDtypeStruct((1, NP_P), _f32),
            jax.ShapeDtypeStruct((1, NP_P), _f32),
            jax.ShapeDtypeStruct((1, NC_P), _f32),
        ],
    )(cat_p, pri_p, mpc_row, mcp_row)


def _gate_body(has_adj, *refs):
    if has_adj:
        (e1_ref, e2_ref, e3_ref, wa_ref, w1_ref, w2_ref, ba_ref, b1_ref,
         b2_ref, a0_ref, a1_ref, o_ref) = refs
    else:
        (e1_ref, e2_ref, e3_ref, wa_ref, w1_ref, w2_ref, ba_ref, b1_ref,
         b2_ref, o_ref) = refs
    x1, x2, x3 = e1_ref[...], e2_ref[...], e3_ref[...]
    wa = wa_ref[...]
    g = (jnp.dot(x1, wa[0:EMB], preferred_element_type=_f32)
         + jnp.dot(x2, wa[EMB:2 * EMB] + w1_ref[...],
                   preferred_element_type=_f32)
         + jnp.dot(x3, wa[2 * EMB:3 * EMB] + w2_ref[...],
                   preferred_element_type=_f32)
         + ba_ref[...] + b1_ref[...] + b2_ref[...])
    gate = jax.nn.sigmoid(g)
    out = x1 + gate * x2 + (1.0 - gate) * x3
    if has_adj:
        out = out + a0_ref[...] + a1_ref[...]
    o_ref[...] = out


def _gate(e1, e2, e3, wa, w1, w2, ba, b1, b2, adj01=None, blk=1024):
    rows = e1.shape[0]
    grid = rows // blk
    has_adj = adj01 is not None
    full = lambda k: (0, 0)
    rspec = pl.BlockSpec((blk, EMB), lambda k: (k, 0))
    in_specs = [rspec, rspec, rspec,
                pl.BlockSpec((3 * EMB, EMB), full),
                pl.BlockSpec((EMB, EMB), full),
                pl.BlockSpec((EMB, EMB), full),
                pl.BlockSpec((1, EMB), full),
                pl.BlockSpec((1, EMB), full),
                pl.BlockSpec((1, EMB), full)]
    args = [e1, e2, e3, wa, w1, w2, ba.reshape(1, EMB), b1.reshape(1, EMB),
            b2.reshape(1, EMB)]
    if has_adj:
        in_specs += [rspec, rspec]
        args += list(adj01)
    return pl.pallas_call(
        functools.partial(_gate_body, has_adj),
        grid=(grid,),
        in_specs=in_specs,
        out_specs=rspec,
        out_shape=jax.ShapeDtypeStruct((rows, EMB), _f32),
    )(*args)


def _pack_edges(er, ec, ev, total):
    e = er.shape[0]
    pad = total - e
    er = jnp.concatenate([er.astype(_i32), jnp.zeros((pad,), _i32)])
    ec = jnp.concatenate([ec.astype(_i32), jnp.zeros((pad,), _i32)])
    ev = jnp.concatenate([lax.bitcast_convert_type(ev, _i32),
                          jnp.zeros((pad,), _i32)])
    packed = jnp.stack([er, ec, ev], axis=0)
    packed = packed.reshape(3, total // ECH, ECH)
    return packed.transpose(1, 0, 2)


def _pad_rows(x, rows):
    return jnp.pad(x, ((0, rows - x.shape[0]), (0, 0)))


def _pad_vec(x, n):
    return jnp.pad(x, (0, n - x.shape[0]))


def kernel(adj_row, adj_col, adj_val, vp_row, vp_col, vp_val, pv_row, pv_col,
           pv_val, vc_row, vc_col, vc_val, cv_row, cv_col, cv_val, pc_row,
           pc_col, pc_val, cp_row, cp_col, cp_val, embedding, pri_emb,
           cate_emb, params):
    p = params
    adj_e = _pack_edges(adj_row, adj_col, adj_val, ADJ_TOT)
    vp_e = _pack_edges(vp_row, vp_col, vp_val, V_TOT)
    vc_e = _pack_edges(vc_row, vc_col, vc_val, V_TOT)
    pv_e = _pack_edges(pv_row, pv_col, pv_val, GA_TOT)
    cv_e = _pack_edges(cv_row, cv_col, cv_val, GA_TOT)
    pc_e = _pack_edges(pc_row, pc_col, pc_val, GB_TOT)
    cp_e = _pack_edges(cp_row, cp_col, cp_val, GB_TOT)

    m_pv = _pad_vec(p["mat_pv"][:, 0], NP_P)
    m_pc = _pad_vec(p["mat_pc"][:, 0], NP_P)
    m_cp = _pad_vec(p["mat_cp"][:, 0], NC_P)
    m_cv = _pad_vec(p["mat_cv"][:, 0], NC_P)
    zeros_t = jnp.zeros((DPT, NN_P), _f32)
    zeros_g = jnp.zeros((DPT, NC_P), _f32)

    emb_c = _pad_rows(embedding, NN_P)
    pri_c = _pad_rows(pri_emb, NP_P)
    cat_c = _pad_rows(cate_emb, NC_P)

    for layer in range(2):
        include_cat = layer == 0
        s_emb2, zpv, zcv = _prep_a(emb_c, m_pv.reshape(1, NP_P),
                                   m_cv.reshape(1, NC_P))
        scat2, spri2, zpc, zcp = _prep_b(cat_c, pri_c, m_pc.reshape(1, NP_P),
                                         m_cp.reshape(1, NC_P))
        sc = _make_sc_kernel(include_cat)
        outs = sc(adj_e, vp_e, vc_e, pv_e, pc_e, cp_e, cv_e,
                  emb_c.T, pri_c.T, cat_c.T,
                  s_emb2.reshape(NN_P), scat2.reshape(NC_P),
                  spri2.reshape(NP_P),
                  zpv.reshape(NP_P), zpc.reshape(NP_P), zcp.reshape(NC_P),
                  zcv.reshape(NC_P),
                  m_pv, m_pc, m_cp, m_cv, zeros_t, zeros_g)
        if include_cat:
            adjT, vpT, vcT, pvT, pcT, cpT, cvT = outs
        else:
            adjT, vpT, vcT, pvT, pcT = outs

        new_emb = _gate(emb_c, vpT.T, vcT.T, p["W_aog_i"], p["W_bog1_i"],
                        p["W_bog2_i"], p["b_aog_i"], p["b_bog1_i"],
                        p["b_bog2_i"],
                        adj01=(adjT[:EMB].T, adjT[EMB:].T), blk=1024)
        new_pri = _gate(pri_c, pvT.T[:NP_P], pcT.T[:NP_P], p["W_aog_p"],
                        p["W_bog1_p"], p["W_bog2_p"], p["b_aog_p"],
                        p["b_bog1_p"], p["b_bog2_p"], blk=NP_P)
        if include_cat:
            cat_c = _gate(cat_c, cpT.T, cvT.T, p["W_aog_c"], p["W_bog1_c"],
                          p["W_bog2_c"], p["b_aog_c"], p["b_bog1_c"],
                          p["b_bog2_c"], blk=NC_P)
        emb_c, pri_c = new_emb, new_pri

    return emb_c[:N_NODE], pri_c[:N_PRICE]


# speedup vs baseline: 1.0498x; 1.0498x over previous
import functools

import jax
import jax.numpy as jnp
from jax import lax
from jax.experimental import pallas as pl
from jax.experimental.pallas import tpu as pltpu
from jax.experimental.pallas import tpu_sc as plsc

N_NODE = 10000
N_PRICE = 100
N_CAT = 1000
EMB = 64

NN_P = 10240
NP_P = 128
NC_P = 1024

NCORES = 2
NSUB = 16
DPT = EMB // NSUB
ECH = 512

ADJ_PC = 626
ADJ_TOT = ADJ_PC * ECH * NCORES
V_PC = 20
V_TOT = V_PC * ECH
GA_PC = 20
GA_TOT = GA_PC * ECH
GB_PC = 4
GB_TOT = GB_PC * ECH

_f32 = jnp.float32
_i32 = jnp.int32


def _sc_body(include_cat, *refs):
    names = [
        "adj_e", "vp_e", "vc_e", "pv_e", "pc_e", "cp_e", "cv_e",
        "emb_t", "pri_t", "cat_t",
        "s_emb", "s_cat", "s_pri", "z_pv", "z_pc", "z_cp", "z_cv", "m_pv",
        "m_pc", "m_cp", "m_cv", "zeros_t", "zeros_g",
        "adjT_out", "vpT_out", "vcT_out", "pvT_out", "pcT_out",
    ] + (["cpT_out", "cvT_out"] if include_cat else []) + [
        "ebuf", "embT_sl", "priT_sl", "catT_sl", "outT", "outG",
        "sbufA", "sbufB", "mbufA", "mbufB", "zbufA", "zbufB",
        "denA", "denB", "wbufA", "wbufB", "esem0", "esem1",
    ]
    r = dict(zip(names, refs, strict=True))

    cid = lax.axis_index("c")
    sid = lax.axis_index("s")

    ebuf = r["ebuf"]
    outT, outG = r["outT"], r["outG"]
    esem = (r["esem0"], r["esem1"])
    dslice = pl.ds(sid * DPT, DPT)

    def e_start(edata, ch, b):
        pltpu.async_copy(edata.at[ch], ebuf.at[b], esem[b])

    def e_wait(edata, ch, b):
        pltpu.make_async_copy(edata.at[ch], ebuf.at[b], esem[b]).wait()

    def edge_loop(edata, cbase, pc, group_fn, unroll):
        for b in range(2):
            e_start(edata, cbase + b, b)

        def body(kk, carry):
            for b in range(2):
                k = kk * 2 + b
                e_wait(edata, cbase + k, b)

                @plsc.parallel_loop(0, ECH // 16, unroll=unroll)
                def _groups(g, b=b, k=k):
                    group_fn(b, k, g)

                @pl.when(k + 2 < pc)
                def _prefetch(b=b, k=k):
                    e_start(edata, cbase + k + 2, b)
            return carry
        lax.fori_loop(0, pc // 2, body, 0)

    def spmm_group(tbl_sl):
        def fn(b, k, g):
            sl = pl.ds(g * 16, 16)
            r16 = ebuf[b, 0, sl]
            c16 = ebuf[b, 1, sl]
            v16 = plsc.bitcast(ebuf[b, 2, sl], _f32)
            for d in range(DPT):
                df = jnp.full((16,), d, _i32)
                gv = plsc.load_gather(tbl_sl, [df, c16])
                plsc.addupdate_scatter(outT, [df, r16], gv * v16)
        return fn

    def weight_group(mref, sref, wbuf, den):
        def fn(b, k, g):
            sl = pl.ds(g * 16, 16)
            r16 = ebuf[b, 0, sl]
            c16 = ebuf[b, 1, sl]
            v16 = plsc.bitcast(ebuf[b, 2, sl], _f32)
            m16 = plsc.load_gather(mref, [r16])
            s16 = plsc.load_gather(sref, [c16])
            w16 = jnp.exp(m16 * s16) * v16
            wbuf[pl.ds(k * ECH + g * 16, 16)] = w16
            plsc.addupdate_scatter(den, [r16], w16)
        return fn

    def gate_group(tbl_sl, wbuf, den, zref):
        def fn(b, k, g):
            sl = pl.ds(g * 16, 16)
            r16 = ebuf[b, 0, sl]
            c16 = ebuf[b, 1, sl]
            w16 = wbuf[pl.ds(k * ECH + g * 16, 16)]
            d16 = plsc.load_gather(den, [r16])
            z16 = plsc.load_gather(zref, [r16])
            u16 = w16 / (d16 + 1e-8 * z16)
            for d in range(DPT):
                df = jnp.full((16,), d, _i32)
                gv = plsc.load_gather(tbl_sl, [df, c16])
                plsc.addupdate_scatter(outG, [df, r16], gv * u16)
        return fn

    def zero_vec(ref, n):
        def zb(i, c):
            ref[pl.ds(i * 16, 16)] = jnp.zeros((16,), _f32)
            return c
        lax.fori_loop(0, n // 16, zb, 0)

    pltpu.sync_copy(r["emb_t"].at[dslice], r["embT_sl"])
    pltpu.sync_copy(r["pri_t"].at[dslice], r["priT_sl"])
    pltpu.sync_copy(r["cat_t"].at[dslice], r["catT_sl"])
    pltpu.sync_copy(r["s_emb"], r["sbufA"])

    @pl.when(cid == 0)
    def _stage0():
        pltpu.sync_copy(r["s_cat"], r["sbufB"])
        pltpu.sync_copy(r["m_pv"], r["mbufA"].at[pl.ds(0, NP_P)])
        pltpu.sync_copy(r["m_pc"], r["mbufB"].at[pl.ds(0, NP_P)])
        pltpu.sync_copy(r["z_pv"], r["zbufA"].at[pl.ds(0, NP_P)])
        pltpu.sync_copy(r["z_pc"], r["zbufB"].at[pl.ds(0, NP_P)])

    @pl.when(cid == 1)
    def _stage1():
        pltpu.sync_copy(r["s_pri"], r["sbufB"].at[pl.ds(0, NP_P)])
        pltpu.sync_copy(r["m_cv"], r["mbufA"])
        pltpu.sync_copy(r["m_cp"], r["mbufB"])
        pltpu.sync_copy(r["z_cv"], r["zbufA"])
        pltpu.sync_copy(r["z_cp"], r["zbufB"])

    pltpu.sync_copy(r["zeros_t"], outT)
    edge_loop(r["adj_e"], cid * ADJ_PC, ADJ_PC, spmm_group(r["embT_sl"]),
              unroll=4)
    pltpu.sync_copy(outT, r["adjT_out"].at[pl.ds(cid * EMB + sid * DPT, DPT)])

    pltpu.sync_copy(r["zeros_t"], outT)

    @pl.when(cid == 0)
    def _vp():
        edge_loop(r["vp_e"], 0, V_PC, spmm_group(r["priT_sl"]), unroll=2)
        pltpu.sync_copy(outT, r["vpT_out"].at[dslice])

    @pl.when(cid == 1)
    def _vc():
        edge_loop(r["vc_e"], 0, V_PC, spmm_group(r["catT_sl"]), unroll=2)
        pltpu.sync_copy(outT, r["vcT_out"].at[dslice])

    zero_vec(r["denA"], NC_P)
    zero_vec(r["denB"], NC_P)

    @pl.when(cid == 0)
    def _gates0():
        edge_loop(r["pv_e"], 0, GA_PC,
                  weight_group(r["mbufA"], r["sbufA"], r["wbufA"], r["denA"]),
                  unroll=1)
        edge_loop(r["pc_e"], 0, GB_PC,
                  weight_group(r["mbufB"], r["sbufB"], r["wbufB"], r["denB"]),
                  unroll=1)
        pltpu.sync_copy(r["zeros_g"], outG)
        edge_loop(r["pv_e"], 0, GA_PC,
                  gate_group(r["embT_sl"], r["wbufA"], r["denA"], r["zbufA"]),
                  unroll=1)
        pltpu.sync_copy(outG, r["pvT_out"].at[dslice])
        pltpu.sync_copy(r["zeros_g"], outG)
        edge_loop(r["pc_e"], 0, GB_PC,
                  gate_group(r["catT_sl"], r["wbufB"], r["denB"], r["zbufB"]),
                  unroll=1)
        pltpu.sync_copy(outG, r["pcT_out"].at[dslice])

    if include_cat:
        @pl.when(cid == 1)
        def _gates1():
            edge_loop(r["cv_e"], 0, GA_PC,
                      weight_group(r["mbufA"], r["sbufA"], r["wbufA"],
                                   r["denA"]), unroll=1)
            edge_loop(r["cp_e"], 0, GB_PC,
                      weight_group(r["mbufB"], r["sbufB"], r["wbufB"],
                                   r["denB"]), unroll=1)
            pltpu.sync_copy(r["zeros_g"], outG)
            edge_loop(r["cv_e"], 0, GA_PC,
                      gate_group(r["embT_sl"], r["wbufA"], r["denA"],
                                 r["zbufA"]), unroll=1)
            pltpu.sync_copy(outG, r["cvT_out"].at[dslice])
            pltpu.sync_copy(r["zeros_g"], outG)
            edge_loop(r["cp_e"], 0, GB_PC,
                      gate_group(r["priT_sl"], r["wbufB"], r["denB"],
                                 r["zbufB"]), unroll=1)
            pltpu.sync_copy(outG, r["cpT_out"].at[dslice])


@functools.cache
def _make_sc_kernel(include_cat: bool):
    out_type = [
        jax.ShapeDtypeStruct((2 * EMB, NN_P), _f32),
        jax.ShapeDtypeStruct((EMB, NN_P), _f32),
        jax.ShapeDtypeStruct((EMB, NN_P), _f32),
        jax.ShapeDtypeStruct((EMB, NC_P), _f32),
        jax.ShapeDtypeStruct((EMB, NC_P), _f32),
    ]
    if include_cat:
        out_type += [
            jax.ShapeDtypeStruct((EMB, NC_P), _f32),
            jax.ShapeDtypeStruct((EMB, NC_P), _f32),
        ]
    scratch = [
        pltpu.VMEM((2, 3, ECH), _i32),
        pltpu.VMEM((DPT, NN_P), _f32),
        pltpu.VMEM((DPT, NP_P), _f32),
        pltpu.VMEM((DPT, NC_P), _f32),
        pltpu.VMEM((DPT, NN_P), _f32),
        pltpu.VMEM((DPT, NC_P), _f32),
        pltpu.VMEM((NN_P,), _f32),
        pltpu.VMEM((NC_P,), _f32),
        pltpu.VMEM((NC_P,), _f32),
        pltpu.VMEM((NC_P,), _f32),
        pltpu.VMEM((NC_P,), _f32),
        pltpu.VMEM((NC_P,), _f32),
        pltpu.VMEM((NC_P,), _f32),
        pltpu.VMEM((NC_P,), _f32),
        pltpu.VMEM((GA_TOT,), _f32),
        pltpu.VMEM((GB_TOT,), _f32),
        pltpu.SemaphoreType.DMA,
        pltpu.SemaphoreType.DMA,
    ]
    mesh = plsc.VectorSubcoreMesh(
        core_axis_name="c", subcore_axis_name="s",
        num_cores=NCORES, num_subcores=NSUB)
    return pl.kernel(
        functools.partial(_sc_body, include_cat),
        out_type=out_type, mesh=mesh, scratch_types=scratch,
        compiler_params=pltpu.CompilerParams(needs_layout_passes=False,
                                             use_tc_tiling_on_sc=False),
        name=f"cohhn_sc_layer_cat{int(include_cat)}")


def _prep_a_body(emb_ref, mpv_ref, mcv_ref, s_ref, zpv_ref, zcv_ref):
    k = pl.program_id(0)
    blk = emb_ref[...]
    s = jnp.sum(blk, axis=1, keepdims=True)
    s_ref[...] = s.reshape(8, 128)
    jrow = lax.broadcasted_iota(_i32, (1024, 1), 0) + k * 1024
    mask = (jrow < N_NODE).astype(_f32)

    @pl.when(k == 0)
    def _init():
        zpv_ref[...] = jnp.zeros_like(zpv_ref)
        zcv_ref[...] = jnp.zeros_like(zcv_ref)

    zpv_ref[...] += jnp.sum(jnp.exp(s * mpv_ref[...]) * mask, axis=0,
                            keepdims=True)
    zcv_ref[...] += jnp.sum(jnp.exp(s * mcv_ref[...]) * mask, axis=0,
                            keepdims=True)


def _prep_a(emb_p, mpv_row, mcv_row):
    return pl.pallas_call(
        _prep_a_body,
        grid=(NN_P // 1024,),
        in_specs=[
            pl.BlockSpec((1024, EMB), lambda k: (k, 0)),
            pl.BlockSpec((1, NP_P), lambda k: (0, 0)),
            pl.BlockSpec((1, NC_P), lambda k: (0, 0)),
        ],
        out_specs=[
            pl.BlockSpec((8, 128), lambda k: (k, 0)),
            pl.BlockSpec((1, NP_P), lambda k: (0, 0)),
            pl.BlockSpec((1, NC_P), lambda k: (0, 0)),
        ],
        out_shape=[
            jax.ShapeDtypeStruct((NN_P // 128, 128), _f32),
            jax.ShapeDtypeStruct((1, NP_P), _f32),
            jax.ShapeDtypeStruct((1, NC_P), _f32),
        ],
    )(emb_p, mpv_row, mcv_row)


def _prep_b_body(cat_ref, pri_ref, mpc_ref, mcp_ref,
                 scat_ref, spri_ref, zpc_ref, zcp_ref):
    scat = jnp.sum(cat_ref[...], axis=1, keepdims=True)
    spri = jnp.sum(pri_ref[...], axis=1, keepdims=True)
    scat_ref[...] = scat.reshape(8, 128)
    spri_ref[...] = spri.reshape(1, 128)
    mc = (lax.broadcasted_iota(_i32, (1024, 1), 0) < N_CAT).astype(_f32)
    mp = (lax.broadcasted_iota(_i32, (128, 1), 0) < N_PRICE).astype(_f32)
    zpc_ref[...] = jnp.sum(jnp.exp(scat * mpc_ref[...]) * mc, axis=0,
                           keepdims=True)
    zcp_ref[...] = jnp.sum(jnp.exp(spri * mcp_ref[...]) * mp, axis=0,
                           keepdims=True)


def _prep_b(cat_p, pri_p, mpc_row, mcp_row):
    return pl.pallas_call(
        _prep_b_body,
        out_shape=[
            jax.ShapeDtypeStruct((NC_P // 128, 128), _f32),
            jax.ShapeDtypeStruct((1, NP_P), _f32),
            jax.ShapeDtypeStruct((1, NP_P), _f32),
            jax.ShapeDtypeStruct((1, NC_P), _f32),
        ],
    )(cat_p, pri_p, mpc_row, mcp_row)


def _gate_body(has_adj, dual, *refs):
    n = 9 + (2 if has_adj else 0)
    (e1_ref, e2t_ref, e3t_ref, wa_ref, w1_ref, w2_ref, ba_ref, b1_ref,
     b2_ref) = refs[:9]
    x1 = e1_ref[...]
    x2 = e2t_ref[...].T
    x3 = e3t_ref[...].T
    wa = wa_ref[...]
    g = (jnp.dot(x1, wa[0:EMB], preferred_element_type=_f32)
         + jnp.dot(x2, wa[EMB:2 * EMB] + w1_ref[...],
                   preferred_element_type=_f32)
         + jnp.dot(x3, wa[2 * EMB:3 * EMB] + w2_ref[...],
                   preferred_element_type=_f32)
         + ba_ref[...] + b1_ref[...] + b2_ref[...])
    gate = jax.nn.sigmoid(g)
    out = x1 + gate * x2 + (1.0 - gate) * x3
    if has_adj:
        out = out + refs[9][...].T + refs[10][...].T
    refs[n][...] = out
    if dual:
        refs[n + 1][...] = out.T


def _gate(e1, e2t, e3t, wa, w1, w2, ba, b1, b2, adjt=None, blk=1024,
          dual=False):
    rows = e1.shape[0]
    grid = rows // blk
    has_adj = adjt is not None
    full = lambda k: (0, 0)
    rspec = pl.BlockSpec((blk, EMB), lambda k: (k, 0))
    tspec = pl.BlockSpec((EMB, blk), lambda k: (0, k))
    in_specs = [rspec, tspec, tspec,
                pl.BlockSpec((3 * EMB, EMB), full),
                pl.BlockSpec((EMB, EMB), full),
                pl.BlockSpec((EMB, EMB), full),
                pl.BlockSpec((1, EMB), full),
                pl.BlockSpec((1, EMB), full),
                pl.BlockSpec((1, EMB), full)]
    args = [e1, e2t, e3t, wa, w1, w2, ba.reshape(1, EMB), b1.reshape(1, EMB),
            b2.reshape(1, EMB)]
    if has_adj:
        in_specs += [pl.BlockSpec((EMB, blk), lambda k: (0, k)),
                     pl.BlockSpec((EMB, blk), lambda k: (1, k))]
        args += [adjt, adjt]
    out_specs = [rspec]
    out_shape = [jax.ShapeDtypeStruct((rows, EMB), _f32)]
    if dual:
        out_specs.append(tspec)
        out_shape.append(jax.ShapeDtypeStruct((EMB, rows), _f32))
    res = pl.pallas_call(
        functools.partial(_gate_body, has_adj, dual),
        grid=(grid,),
        in_specs=in_specs,
        out_specs=out_specs,
        out_shape=out_shape,
    )(*args)
    return res if dual else (res[0], None)


def _pack_edges(er, ec, ev, total):
    e = er.shape[0]
    pad = total - e
    er = jnp.concatenate([er.astype(_i32), jnp.zeros((pad,), _i32)])
    ec = jnp.concatenate([ec.astype(_i32), jnp.zeros((pad,), _i32)])
    ev = jnp.concatenate([lax.bitcast_convert_type(ev, _i32),
                          jnp.zeros((pad,), _i32)])
    packed = jnp.stack([er, ec, ev], axis=0)
    packed = packed.reshape(3, total // ECH, ECH)
    return packed.transpose(1, 0, 2)


def _pad_rows(x, rows):
    return jnp.pad(x, ((0, rows - x.shape[0]), (0, 0)))


def _pad_vec(x, n):
    return jnp.pad(x, (0, n - x.shape[0]))


def kernel(adj_row, adj_col, adj_val, vp_row, vp_col, vp_val, pv_row, pv_col,
           pv_val, vc_row, vc_col, vc_val, cv_row, cv_col, cv_val, pc_row,
           pc_col, pc_val, cp_row, cp_col, cp_val, embedding, pri_emb,
           cate_emb, params):
    p = params
    adj_e = _pack_edges(adj_row, adj_col, adj_val, ADJ_TOT)
    vp_e = _pack_edges(vp_row, vp_col, vp_val, V_TOT)
    vc_e = _pack_edges(vc_row, vc_col, vc_val, V_TOT)
    pv_e = _pack_edges(pv_row, pv_col, pv_val, GA_TOT)
    cv_e = _pack_edges(cv_row, cv_col, cv_val, GA_TOT)
    pc_e = _pack_edges(pc_row, pc_col, pc_val, GB_TOT)
    cp_e = _pack_edges(cp_row, cp_col, cp_val, GB_TOT)

    m_pv = _pad_vec(p["mat_pv"][:, 0], NP_P)
    m_pc = _pad_vec(p["mat_pc"][:, 0], NP_P)
    m_cp = _pad_vec(p["mat_cp"][:, 0], NC_P)
    m_cv = _pad_vec(p["mat_cv"][:, 0], NC_P)
    zeros_t = jnp.zeros((DPT, NN_P), _f32)
    zeros_g = jnp.zeros((DPT, NC_P), _f32)

    emb_c = _pad_rows(embedding, NN_P)
    pri_c = _pad_rows(pri_emb, NP_P)
    cat_c = _pad_rows(cate_emb, NC_P)
    emb_t, pri_t, cat_t = emb_c.T, pri_c.T, cat_c.T

    for layer in range(2):
        include_cat = layer == 0
        s_emb2, zpv, zcv = _prep_a(emb_c, m_pv.reshape(1, NP_P),
                                   m_cv.reshape(1, NC_P))
        scat2, spri2, zpc, zcp = _prep_b(cat_c, pri_c, m_pc.reshape(1, NP_P),
                                         m_cp.reshape(1, NC_P))
        sc = _make_sc_kernel(include_cat)
        outs = sc(adj_e, vp_e, vc_e, pv_e, pc_e, cp_e, cv_e,
                  emb_t, pri_t, cat_t,
                  s_emb2.reshape(NN_P), scat2.reshape(NC_P),
                  spri2.reshape(NP_P),
                  zpv.reshape(NP_P), zpc.reshape(NP_P), zcp.reshape(NC_P),
                  zcv.reshape(NC_P),
                  m_pv, m_pc, m_cp, m_cv, zeros_t, zeros_g)
        if include_cat:
            adjT, vpT, vcT, pvT, pcT, cpT, cvT = outs
        else:
            adjT, vpT, vcT, pvT, pcT = outs

        dual = include_cat
        new_emb, new_embt = _gate(emb_c, vpT, vcT, p["W_aog_i"],
                                  p["W_bog1_i"], p["W_bog2_i"], p["b_aog_i"],
                                  p["b_bog1_i"], p["b_bog2_i"],
                                  adjt=adjT, blk=1024, dual=dual)
        new_pri, new_prit = _gate(pri_c, pvT, pcT, p["W_aog_p"],
                                  p["W_bog1_p"], p["W_bog2_p"], p["b_aog_p"],
                                  p["b_bog1_p"], p["b_bog2_p"], blk=NP_P,
                                  dual=dual)
        if include_cat:
            cat_c, cat_t = _gate(cat_c, cpT, cvT, p["W_aog_c"],
                                 p["W_bog1_c"], p["W_bog2_c"], p["b_aog_c"],
                                 p["b_bog1_c"], p["b_bog2_c"], blk=NC_P,
                                 dual=True)
        emb_c, pri_c, emb_t, pri_t = new_emb, new_pri, new_embt, new_prit

    return emb_c[:N_NODE], pri_c[:N_PRICE]
